# Initial kernel scaffold; baseline (speedup 1.0000x reference)
import jax
import jax.numpy as jnp
from jax.experimental import pallas as pl


def kernel(inputs, sf_W1, sf_b1, sf_W2, sf_b2, sf_Wo, sf_bo, out_W1, out_b1, out_W2, out_b2, out_Wo, out_bo):
    raise NotImplementedError("write your pallas kernel here")


# speedup vs baseline: 6.9167x; 6.9167x over previous
import functools

import jax
import jax.numpy as jnp
from jax.experimental import pallas as pl

P = 64
KTOP = 16
GRID_WD = 8
G = 16


def _body(x_ref, w1_ref, b1_ref, w2_ref, b2_ref, wo_ref, bo_ref,
          ow1x_ref, ow1s_ref, ob1_ref, ow2_ref, ob2_ref, owo_ref, obo_ref,
          out_ref):
    R = G * P
    f32 = jnp.float32

    s_blocks = []
    for g in range(G):
        xg = x_ref[g]
        s_blocks.append(jax.lax.dot_general(
            xg, xg, (((1,), (1,)), ((), ())), preferred_element_type=f32))
    S = jnp.concatenate(s_blocks, axis=0)

    col = jax.lax.broadcasted_iota(jnp.int32, (R, P), 1)
    neg = f32(-jnp.inf)
    vals, idxs = [], []
    for _ in range(KTOP):
        m = jnp.max(S, axis=1, keepdims=True)
        cand = jnp.where(S == m, col, P)
        a = jnp.min(cand, axis=1, keepdims=True)
        vals.append(m)
        idxs.append(a)
        S = jnp.where(col == a, neg, S)
    V = jnp.concatenate(vals, axis=1)
    I = jnp.concatenate(idxs, axis=1)

    p = jax.lax.broadcasted_iota(jnp.int32, (R, KTOP), 0) % P
    inv = f32(1.0 / (GRID_WD - 1))
    dh = (p // GRID_WD - I // GRID_WD).astype(f32) * inv
    dw = (p % GRID_WD - I % GRID_WD).astype(f32) * inv

    feat = jnp.concatenate([V, dh, dw], axis=1)
    h = jnp.maximum(jnp.dot(feat, w1_ref[...], preferred_element_type=f32)
                    + b1_ref[...], 0.0)
    h = jnp.maximum(jnp.dot(h, w2_ref[...], preferred_element_type=f32)
                    + b2_ref[...], 0.0)
    sf = jnp.dot(h, wo_ref[...], preferred_element_type=f32) + bo_ref[...]

    x2d = x_ref[...].reshape(R, x_ref.shape[2])
    y = jnp.dot(x2d, ow1x_ref[...], preferred_element_type=f32)
    y = y + jnp.dot(sf, ow1s_ref[...], preferred_element_type=f32)
    y = jnp.maximum(y + ob1_ref[...], 0.0)
    y = jnp.maximum(jnp.dot(y, ow2_ref[...], preferred_element_type=f32)
                    + ob2_ref[...], 0.0)
    out_ref[...] = (jnp.dot(y, owo_ref[...], preferred_element_type=f32)
                    + obo_ref[...])


def _forward(inputs, sf_W1, sf_b1, sf_W2, sf_b2, sf_Wo, sf_bo,
             out_W1, out_b1, out_W2, out_b2, out_Wo, out_bo,
             interpret=False):
    B, Wn, P_, C = inputs.shape
    NW = B * Wn
    x = inputs.reshape(NW, P_, C)

    w1p = jnp.concatenate([sf_W1[0:KTOP],
                           sf_W1[KTOP::2],
                           sf_W1[KTOP + 1::2]], axis=0)
    ow1x = out_W1[:C]
    ow1s = out_W1[C:]

    row2 = lambda b: b.reshape(1, -1)
    weights = (w1p, row2(sf_b1), sf_W2, row2(sf_b2), sf_Wo, row2(sf_bo),
               ow1x, ow1s, row2(out_b1), out_W2, row2(out_b2),
               out_Wo, row2(out_bo))

    wspecs = [pl.BlockSpec(w.shape, lambda i: (0, 0)) for w in weights]
    out = pl.pallas_call(
        _body,
        grid=(NW // G,),
        in_specs=[pl.BlockSpec((G, P_, C), lambda i: (i, 0, 0))] + wspecs,
        out_specs=pl.BlockSpec((G * P_, 256), lambda i: (i, 0)),
        out_shape=jax.ShapeDtypeStruct((NW * P_, 256), jnp.float32),
        interpret=interpret,
    )(x, *weights)
    return out.reshape(B, Wn, P_, 256)


def kernel(inputs, sf_W1, sf_b1, sf_W2, sf_b2, sf_Wo, sf_bo,
           out_W1, out_b1, out_W2, out_b2, out_Wo, out_bo):
    return _forward(inputs, sf_W1, sf_b1, sf_W2, sf_b2, sf_Wo, sf_bo,
                    out_W1, out_b1, out_W2, out_b2, out_Wo, out_bo)


# speedup vs baseline: 16.3387x; 2.3622x over previous
import jax
import jax.numpy as jnp
from jax.experimental import pallas as pl

P = 64
KTOP = 16
GRID_WD = 8
G = 16


def _body(x_ref, w1t_ref, b1_ref, w2t_ref, b2_ref, wot_ref, bo_ref,
          ow1x_ref, ow1s_ref, ob1_ref, ow2_ref, ob2_ref, owo_ref, obo_ref,
          out_ref):
    R = G * P
    f32 = jnp.float32

    s_blocks = []
    for g in range(G):
        xg = x_ref[g]
        s_blocks.append(jax.lax.dot_general(
            xg, xg, (((1,), (1,)), ((), ())), preferred_element_type=f32))
    S = jnp.concatenate(s_blocks, axis=1)

    qv = jax.lax.broadcasted_iota(jnp.int32, (1, P), 1).astype(f32)
    neg = f32(-jnp.inf)
    vals, idxs = [], []
    for _ in range(KTOP):
        m = jnp.max(S, axis=0, keepdims=True)
        ismax = (S == m).astype(f32)
        a = jax.lax.dot_general(
            qv, ismax, (((1,), (0,)), ((), ())), preferred_element_type=f32)
        vals.append(m)
        idxs.append(a)
        S = jnp.where(ismax > 0.0, neg, S)
    V = jnp.concatenate(vals, axis=0)
    I = jnp.concatenate(idxs, axis=0)

    lane = jax.lax.broadcasted_iota(jnp.int32, (KTOP, R), 1)
    p = (lane % P).astype(f32)
    ph = jnp.floor(p * 0.125)
    pw = p - 8.0 * ph
    ih = jnp.floor(I * 0.125)
    iw = I - 8.0 * ih
    inv = f32(1.0 / (GRID_WD - 1))
    dh = (ph - ih) * inv
    dw = (pw - iw) * inv

    featT = jnp.concatenate([V, dh, dw], axis=0)
    hT = jnp.maximum(jnp.dot(w1t_ref[...], featT,
                             preferred_element_type=f32) + b1_ref[...], 0.0)
    hT = jnp.maximum(jnp.dot(w2t_ref[...], hT,
                             preferred_element_type=f32) + b2_ref[...], 0.0)
    sfT = jnp.dot(wot_ref[...], hT,
                  preferred_element_type=f32) + bo_ref[...]

    x2d = x_ref[...].reshape(R, x_ref.shape[2])
    y = jnp.dot(x2d, ow1x_ref[...], preferred_element_type=f32)
    y = y + jax.lax.dot_general(
        sfT, ow1s_ref[...], (((0,), (0,)), ((), ())),
        preferred_element_type=f32)
    y = jnp.maximum(y + ob1_ref[...], 0.0)
    y = jnp.maximum(jnp.dot(y, ow2_ref[...], preferred_element_type=f32)
                    + ob2_ref[...], 0.0)
    out_ref[...] = (jnp.dot(y, owo_ref[...], preferred_element_type=f32)
                    + obo_ref[...])


def _forward(inputs, sf_W1, sf_b1, sf_W2, sf_b2, sf_Wo, sf_bo,
             out_W1, out_b1, out_W2, out_b2, out_Wo, out_bo,
             interpret=False):
    B, Wn, P_, C = inputs.shape
    NW = B * Wn
    x = inputs.reshape(NW, P_, C)

    w1p = jnp.concatenate([sf_W1[0:KTOP],
                           sf_W1[KTOP::2],
                           sf_W1[KTOP + 1::2]], axis=0)
    ow1x = out_W1[:C]
    ow1s = out_W1[C:]

    col = lambda b: b.reshape(-1, 1)
    row = lambda b: b.reshape(1, -1)
    weights = (w1p.T, col(sf_b1), sf_W2.T, col(sf_b2), sf_Wo.T, col(sf_bo),
               ow1x, ow1s, row(out_b1), out_W2, row(out_b2),
               out_Wo, row(out_bo))

    wspecs = [pl.BlockSpec(w.shape, lambda i: (0, 0)) for w in weights]
    out = pl.pallas_call(
        _body,
        grid=(NW // G,),
        in_specs=[pl.BlockSpec((G, P_, C), lambda i: (i, 0, 0))] + wspecs,
        out_specs=pl.BlockSpec((G * P_, 256), lambda i: (i, 0)),
        out_shape=jax.ShapeDtypeStruct((NW * P_, 256), jnp.float32),
        interpret=interpret,
    )(x, *weights)
    return out.reshape(B, Wn, P_, 256)


def kernel(inputs, sf_W1, sf_b1, sf_W2, sf_b2, sf_Wo, sf_bo,
           out_W1, out_b1, out_W2, out_b2, out_Wo, out_bo):
    return _forward(inputs, sf_W1, sf_b1, sf_W2, sf_b2, sf_Wo, sf_bo,
                    out_W1, out_b1, out_W2, out_b2, out_Wo, out_bo)


# speedup vs baseline: 16.8345x; 1.0303x over previous
import jax
import jax.numpy as jnp
from jax.experimental import pallas as pl

P = 64
KTOP = 16
GRID_WD = 8
G = 16


def _body(x_ref, w1t_ref, b1_ref, w2t_ref, b2_ref, wot_ref, bo_ref,
          ow1x_ref, ow1s_ref, ob1_ref, ow2_ref, ob2_ref, owo_ref, obo_ref,
          out_ref):
    R = G * P
    f32 = jnp.float32

    s_blocks = []
    for g in range(G):
        xg = x_ref[g]
        s_blocks.append(jax.lax.dot_general(
            xg, xg, (((1,), (1,)), ((), ())), preferred_element_type=f32))
    S = jnp.concatenate(s_blocks, axis=1)

    bf16 = jnp.bfloat16
    qv = jax.lax.broadcasted_iota(jnp.int32, (1, P), 1).astype(bf16)
    neg = f32(-jnp.inf)
    vals, idxs = [], []
    for _ in range(KTOP):
        m = jnp.max(S, axis=0, keepdims=True)
        ismax = S == m
        a = jax.lax.dot_general(
            qv, ismax.astype(bf16), (((1,), (0,)), ((), ())),
            preferred_element_type=f32)
        vals.append(m)
        idxs.append(a)
        S = jnp.where(ismax, neg, S)
    V = jnp.concatenate(vals, axis=0)
    I = jnp.concatenate(idxs, axis=0)

    lane = jax.lax.broadcasted_iota(jnp.int32, (KTOP, R), 1)
    p = (lane % P).astype(f32)
    ph = jnp.floor(p * 0.125)
    pw = p - 8.0 * ph
    ih = jnp.floor(I * 0.125)
    iw = I - 8.0 * ih
    inv = f32(1.0 / (GRID_WD - 1))
    dh = (ph - ih) * inv
    dw = (pw - iw) * inv

    featT = jnp.concatenate([V, dh, dw], axis=0).astype(bf16)
    hT = jnp.maximum(jnp.dot(w1t_ref[...], featT,
                             preferred_element_type=f32) + b1_ref[...], 0.0)
    hT = jnp.maximum(jnp.dot(w2t_ref[...], hT.astype(bf16),
                             preferred_element_type=f32) + b2_ref[...], 0.0)
    sfT = jnp.dot(wot_ref[...], hT.astype(bf16),
                  preferred_element_type=f32) + bo_ref[...]

    x2d = x_ref[...].reshape(R, x_ref.shape[2]).astype(bf16)
    y = jnp.dot(x2d, ow1x_ref[...], preferred_element_type=f32)
    y = y + jax.lax.dot_general(
        sfT.astype(bf16), ow1s_ref[...], (((0,), (0,)), ((), ())),
        preferred_element_type=f32)
    y = jnp.maximum(y + ob1_ref[...], 0.0).astype(bf16)
    y = jnp.maximum(jnp.dot(y, ow2_ref[...], preferred_element_type=f32)
                    + ob2_ref[...], 0.0).astype(bf16)
    out_ref[...] = (jnp.dot(y, owo_ref[...], preferred_element_type=f32)
                    + obo_ref[...])


def _forward(inputs, sf_W1, sf_b1, sf_W2, sf_b2, sf_Wo, sf_bo,
             out_W1, out_b1, out_W2, out_b2, out_Wo, out_bo,
             interpret=False):
    B, Wn, P_, C = inputs.shape
    NW = B * Wn
    x = inputs.reshape(NW, P_, C)

    w1p = jnp.concatenate([sf_W1[0:KTOP],
                           sf_W1[KTOP::2],
                           sf_W1[KTOP + 1::2]], axis=0)
    ow1x = out_W1[:C]
    ow1s = out_W1[C:]

    col = lambda b: b.reshape(-1, 1)
    row = lambda b: b.reshape(1, -1)
    bf = lambda w: w.astype(jnp.bfloat16)
    weights = (bf(w1p.T), col(sf_b1), bf(sf_W2.T), col(sf_b2),
               bf(sf_Wo.T), col(sf_bo),
               bf(ow1x), bf(ow1s), row(out_b1), bf(out_W2), row(out_b2),
               bf(out_Wo), row(out_bo))

    wspecs = [pl.BlockSpec(w.shape, lambda i: (0, 0)) for w in weights]
    out = pl.pallas_call(
        _body,
        grid=(NW // G,),
        in_specs=[pl.BlockSpec((G, P_, C), lambda i: (i, 0, 0))] + wspecs,
        out_specs=pl.BlockSpec((G * P_, 256), lambda i: (i, 0)),
        out_shape=jax.ShapeDtypeStruct((NW * P_, 256), jnp.float32),
        interpret=interpret,
    )(x, *weights)
    return out.reshape(B, Wn, P_, 256)


def kernel(inputs, sf_W1, sf_b1, sf_W2, sf_b2, sf_Wo, sf_bo,
           out_W1, out_b1, out_W2, out_b2, out_Wo, out_bo):
    return _forward(inputs, sf_W1, sf_b1, sf_W2, sf_b2, sf_Wo, sf_bo,
                    out_W1, out_b1, out_W2, out_b2, out_Wo, out_bo)


# speedup vs baseline: 18.6060x; 1.1052x over previous
import jax
import jax.numpy as jnp
from jax.experimental import pallas as pl

P = 64
KTOP = 16
GRID_WD = 8
G = 32


def _body(x_ref, w1t_ref, b1_ref, w2t_ref, b2_ref, wot_ref, bo_ref,
          ow1x_ref, ow1s_ref, ob1_ref, ow2_ref, ob2_ref, owo_ref, obo_ref,
          out_ref):
    R = G * P
    f32 = jnp.float32

    s_blocks = []
    for g in range(G):
        xg = x_ref[g]
        s_blocks.append(jax.lax.dot_general(
            xg, xg, (((1,), (1,)), ((), ())), preferred_element_type=f32))
    S = jnp.concatenate(s_blocks, axis=1)

    bf16 = jnp.bfloat16
    qv = jax.lax.broadcasted_iota(jnp.int32, (1, P), 1).astype(bf16)
    neg = f32(-jnp.inf)
    vals, idxs = [], []
    for _ in range(KTOP):
        m = jnp.max(S, axis=0, keepdims=True)
        ismax = S == m
        a = jax.lax.dot_general(
            qv, ismax.astype(bf16), (((1,), (0,)), ((), ())),
            preferred_element_type=f32)
        vals.append(m)
        idxs.append(a)
        S = jnp.where(ismax, neg, S)
    V = jnp.concatenate(vals, axis=0)
    I = jnp.concatenate(idxs, axis=0)

    lane = jax.lax.broadcasted_iota(jnp.int32, (KTOP, R), 1)
    p = (lane % P).astype(f32)
    ph = jnp.floor(p * 0.125)
    pw = p - 8.0 * ph
    ih = jnp.floor(I * 0.125)
    iw = I - 8.0 * ih
    inv = f32(1.0 / (GRID_WD - 1))
    dh = (ph - ih) * inv
    dw = (pw - iw) * inv

    featT = jnp.concatenate([V, dh, dw], axis=0).astype(bf16)
    hT = jnp.maximum(jnp.dot(w1t_ref[...], featT,
                             preferred_element_type=f32) + b1_ref[...], 0.0)
    hT = jnp.maximum(jnp.dot(w2t_ref[...], hT.astype(bf16),
                             preferred_element_type=f32) + b2_ref[...], 0.0)
    sfT = jnp.dot(wot_ref[...], hT.astype(bf16),
                  preferred_element_type=f32) + bo_ref[...]

    x2d = x_ref[...].reshape(R, x_ref.shape[2]).astype(bf16)
    y = jnp.dot(x2d, ow1x_ref[...], preferred_element_type=f32)
    y = y + jax.lax.dot_general(
        sfT.astype(bf16), ow1s_ref[...], (((0,), (0,)), ((), ())),
        preferred_element_type=f32)
    y = jnp.maximum(y + ob1_ref[...], 0.0).astype(bf16)
    y = jnp.maximum(jnp.dot(y, ow2_ref[...], preferred_element_type=f32)
                    + ob2_ref[...], 0.0).astype(bf16)
    out_ref[...] = (jnp.dot(y, owo_ref[...], preferred_element_type=f32)
                    + obo_ref[...])


def _forward(inputs, sf_W1, sf_b1, sf_W2, sf_b2, sf_Wo, sf_bo,
             out_W1, out_b1, out_W2, out_b2, out_Wo, out_bo,
             interpret=False):
    B, Wn, P_, C = inputs.shape
    NW = B * Wn
    x = inputs.reshape(NW, P_, C)

    w1p = jnp.concatenate([sf_W1[0:KTOP],
                           sf_W1[KTOP::2],
                           sf_W1[KTOP + 1::2]], axis=0)
    ow1x = out_W1[:C]
    ow1s = out_W1[C:]

    col = lambda b: b.reshape(-1, 1)
    row = lambda b: b.reshape(1, -1)
    bf = lambda w: w.astype(jnp.bfloat16)
    weights = (bf(w1p.T), col(sf_b1), bf(sf_W2.T), col(sf_b2),
               bf(sf_Wo.T), col(sf_bo),
               bf(ow1x), bf(ow1s), row(out_b1), bf(out_W2), row(out_b2),
               bf(out_Wo), row(out_bo))

    wspecs = [pl.BlockSpec(w.shape, lambda i: (0, 0)) for w in weights]
    out = pl.pallas_call(
        _body,
        grid=(NW // G,),
        in_specs=[pl.BlockSpec((G, P_, C), lambda i: (i, 0, 0))] + wspecs,
        out_specs=pl.BlockSpec((G * P_, 256), lambda i: (i, 0)),
        out_shape=jax.ShapeDtypeStruct((NW * P_, 256), jnp.float32),
        interpret=interpret,
    )(x, *weights)
    return out.reshape(B, Wn, P_, 256)


def kernel(inputs, sf_W1, sf_b1, sf_W2, sf_b2, sf_Wo, sf_bo,
           out_W1, out_b1, out_W2, out_b2, out_Wo, out_bo):
    return _forward(inputs, sf_W1, sf_b1, sf_W2, sf_b2, sf_Wo, sf_bo,
                    out_W1, out_b1, out_W2, out_b2, out_Wo, out_bo)


# speedup vs baseline: 19.1227x; 1.0278x over previous
import jax
import jax.numpy as jnp
from jax.experimental import pallas as pl

P = 64
KTOP = 16
GRID_WD = 8
G = 64


def _body(x_ref, w1t_ref, b1_ref, w2t_ref, b2_ref, wot_ref, bo_ref,
          ow1x_ref, ow1s_ref, ob1_ref, ow2_ref, ob2_ref, owo_ref, obo_ref,
          out_ref):
    R = G * P
    f32 = jnp.float32

    s_blocks = []
    for g in range(G):
        xg = x_ref[g]
        s_blocks.append(jax.lax.dot_general(
            xg, xg, (((1,), (1,)), ((), ())), preferred_element_type=f32))
    S = jnp.concatenate(s_blocks, axis=1)

    bf16 = jnp.bfloat16
    qv = jax.lax.broadcasted_iota(jnp.int32, (1, P), 1).astype(bf16)
    neg = f32(-jnp.inf)
    vals, idxs = [], []
    for _ in range(KTOP):
        m = jnp.max(S, axis=0, keepdims=True)
        ismax = S == m
        a = jax.lax.dot_general(
            qv, ismax.astype(bf16), (((1,), (0,)), ((), ())),
            preferred_element_type=f32)
        vals.append(m)
        idxs.append(a)
        S = jnp.where(ismax, neg, S)
    V = jnp.concatenate(vals, axis=0)
    I = jnp.concatenate(idxs, axis=0)

    lane = jax.lax.broadcasted_iota(jnp.int32, (KTOP, R), 1)
    p = (lane % P).astype(f32)
    ph = jnp.floor(p * 0.125)
    pw = p - 8.0 * ph
    ih = jnp.floor(I * 0.125)
    iw = I - 8.0 * ih
    inv = f32(1.0 / (GRID_WD - 1))
    dh = (ph - ih) * inv
    dw = (pw - iw) * inv

    featT = jnp.concatenate([V, dh, dw], axis=0).astype(bf16)
    hT = jnp.maximum(jnp.dot(w1t_ref[...], featT,
                             preferred_element_type=f32) + b1_ref[...], 0.0)
    hT = jnp.maximum(jnp.dot(w2t_ref[...], hT.astype(bf16),
                             preferred_element_type=f32) + b2_ref[...], 0.0)
    sfT = jnp.dot(wot_ref[...], hT.astype(bf16),
                  preferred_element_type=f32) + bo_ref[...]

    x2d = x_ref[...].reshape(R, x_ref.shape[2]).astype(bf16)
    y = jnp.dot(x2d, ow1x_ref[...], preferred_element_type=f32)
    y = y + jax.lax.dot_general(
        sfT.astype(bf16), ow1s_ref[...], (((0,), (0,)), ((), ())),
        preferred_element_type=f32)
    y = jnp.maximum(y + ob1_ref[...], 0.0).astype(bf16)
    y = jnp.maximum(jnp.dot(y, ow2_ref[...], preferred_element_type=f32)
                    + ob2_ref[...], 0.0).astype(bf16)
    out_ref[...] = (jnp.dot(y, owo_ref[...], preferred_element_type=f32)
                    + obo_ref[...])


def _forward(inputs, sf_W1, sf_b1, sf_W2, sf_b2, sf_Wo, sf_bo,
             out_W1, out_b1, out_W2, out_b2, out_Wo, out_bo,
             interpret=False):
    B, Wn, P_, C = inputs.shape
    NW = B * Wn
    x = inputs.reshape(NW, P_, C)

    w1p = jnp.concatenate([sf_W1[0:KTOP],
                           sf_W1[KTOP::2],
                           sf_W1[KTOP + 1::2]], axis=0)
    ow1x = out_W1[:C]
    ow1s = out_W1[C:]

    col = lambda b: b.reshape(-1, 1)
    row = lambda b: b.reshape(1, -1)
    bf = lambda w: w.astype(jnp.bfloat16)
    weights = (bf(w1p.T), col(sf_b1), bf(sf_W2.T), col(sf_b2),
               bf(sf_Wo.T), col(sf_bo),
               bf(ow1x), bf(ow1s), row(out_b1), bf(out_W2), row(out_b2),
               bf(out_Wo), row(out_bo))

    wspecs = [pl.BlockSpec(w.shape, lambda i: (0, 0)) for w in weights]
    out = pl.pallas_call(
        _body,
        grid=(NW // G,),
        in_specs=[pl.BlockSpec((G, P_, C), lambda i: (i, 0, 0))] + wspecs,
        out_specs=pl.BlockSpec((G * P_, 256), lambda i: (i, 0)),
        out_shape=jax.ShapeDtypeStruct((NW * P_, 256), jnp.float32),
        interpret=interpret,
    )(x, *weights)
    return out.reshape(B, Wn, P_, 256)


def kernel(inputs, sf_W1, sf_b1, sf_W2, sf_b2, sf_Wo, sf_bo,
           out_W1, out_b1, out_W2, out_b2, out_Wo, out_bo):
    return _forward(inputs, sf_W1, sf_b1, sf_W2, sf_b2, sf_Wo, sf_bo,
                    out_W1, out_b1, out_W2, out_b2, out_Wo, out_bo)
